# baseline (speedup 1.0000x reference)
import functools

import jax
import jax.numpy as jnp
from jax.experimental import pallas as pl
from jax.experimental.pallas import tpu as pltpu

N = 2048
M = 65536
K = 4096
RATIO = 0.3
RADIUS2 = 900.0
BLUR = 0.01
EPS = BLUR ** 2
NIT = 5
CHUNK = 2048
NCHUNK = M // CHUNK

_HIGH = jax.lax.Precision.HIGHEST


def _dotT(a, b):
    return jax.lax.dot_general(
        a, b, (((1,), (1,)), ((), ())), precision=_HIGH,
        preferred_element_type=jnp.float32)


def _tri_kernel(pr_ref, mapT_ref, mx_ref, my_ref, mz_ref, pose_ref, out_ref,
                pen1_ref, w2_ref):
    t0 = pose_ref[0, 3]
    t1 = pose_ref[1, 3]
    t2 = pose_ref[2, 3]

    dx = mx_ref[...] - t0
    dy = my_ref[...] - t1
    dz = mz_ref[...] - t2
    d = dx * dx + dy * dy + dz * dz
    bits = jax.lax.bitcast_convert_type(d, jnp.int32)

    def bs_body(_, lohi):
        lo, hi = lohi
        mid = jax.lax.div(lo + hi, 2)
        cnt = jnp.sum((bits <= mid).astype(jnp.int32))
        return jnp.where(cnt >= K, lo, mid + 1), jnp.where(cnt >= K, mid, hi)

    lo0 = jnp.int32(0)
    hi0 = jnp.int32(0x7F800000)
    _, T = jax.lax.fori_loop(0, 31, bs_body, (lo0, hi0))
    c_lt = jnp.sum((bits < T).astype(jnp.int32))
    r_need = K - c_lt

    eq = (bits == T)
    eqf = eq.astype(jnp.float32)
    rowsum = jnp.sum(eqf, axis=1)
    rr = jax.lax.broadcasted_iota(jnp.int32, (512, 512), 0)
    kk = jax.lax.broadcasted_iota(jnp.int32, (512, 512), 1)
    prev_rows = jnp.sum(jnp.where(kk < rr, rowsum[None, :], 0.0), axis=1)
    ci = jax.lax.broadcasted_iota(jnp.int32, (128, 128), 0)
    cj = jax.lax.broadcasted_iota(jnp.int32, (128, 128), 1)
    ltri = (ci < cj).astype(jnp.float32)
    in_row = jax.lax.dot_general(
        eqf, ltri, (((1,), (0,)), ((), ())), precision=_HIGH,
        preferred_element_type=jnp.float32)
    rank = prev_rows[:, None] + in_row
    sel = (bits < T) | (eq & (rank < r_need.astype(jnp.float32)))

    mask = sel & (d <= RADIUS2)
    w2 = mask.astype(jnp.float32)
    pen1 = jnp.where(mask, 0.0, jnp.inf)
    denom = jnp.maximum(jnp.sum(w2), 1.0)

    pen1_ref[...] = jnp.reshape(pen1, (NCHUNK, CHUNK))
    w2_ref[...] = jnp.reshape(w2, (NCHUNK, CHUNK))

    pr = pr_ref[...]
    rn = jnp.sum(pr * pr, axis=1, keepdims=True)

    def chunk_body(i, carry):
        rowmin, l2sum = carry
        mchunk = mapT_ref[:, pl.ds(i * CHUNK, CHUNK)]
        cn = jnp.sum(mchunk * mchunk, axis=0, keepdims=True)
        cross = jax.lax.dot_general(
            pr, mchunk, (((1,), (0,)), ((), ())), precision=_HIGH,
            preferred_element_type=jnp.float32)
        dd = jnp.maximum(rn + cn - 2.0 * cross, 0.0)
        p1 = pen1_ref[pl.ds(i, 1), :]
        rowmin = jnp.minimum(rowmin, jnp.min(dd + p1, axis=1, keepdims=True))
        m2 = jnp.min(dd, axis=0, keepdims=True)
        l2sum = l2sum + jnp.sum(w2_ref[pl.ds(i, 1), :] * m2)
        return rowmin, l2sum

    rowmin0 = jnp.full((N, 1), jnp.inf, dtype=jnp.float32)
    rowmin, l2sum = jax.lax.fori_loop(0, NCHUNK, chunk_body, (rowmin0, 0.0))
    l1 = jnp.sum(rowmin) / N
    out_ref[0, 0] = l1 + l2sum / denom


def _sqd_half(x, y):
    xn = jnp.sum(x * x, axis=1, keepdims=True)
    yn = jnp.sum(y * y, axis=1, keepdims=True)
    ynT = jnp.reshape(yn, (1, -1))
    return 0.5 * jnp.maximum(xn + ynT - 2.0 * _dotT(x, y), 0.0)


def _ot_from_C(C):
    n = C.shape[0]
    inv_eps = 1.0 / EPS
    eps_logn = EPS * jnp.log(jnp.float32(n))
    f = jnp.zeros((n, 1), dtype=jnp.float32)

    def body(_, fg):
        f, _g = fg
        A = (f - C) * inv_eps
        mxc = jnp.max(A, axis=0, keepdims=True)
        g = eps_logn - EPS * (
            mxc + jnp.log(jnp.sum(jnp.exp(A - mxc), axis=0, keepdims=True)))
        B = (g - C) * inv_eps
        mxr = jnp.max(B, axis=1, keepdims=True)
        f = eps_logn - EPS * (
            mxr + jnp.log(jnp.sum(jnp.exp(B - mxr), axis=1, keepdims=True)))
        return f, g

    g0 = jnp.zeros((1, n), dtype=jnp.float32)
    f, g = jax.lax.fori_loop(0, NIT, body, (f, g0))
    return jnp.mean(f) + jnp.mean(g)


def _upsample_kernel(pr_ref, pg_ref, out_ref):
    x = pr_ref[...]
    y = pg_ref[...]
    C_xy = _sqd_half(x, y)
    ot_xy = _ot_from_C(C_xy)
    C_xx = _sqd_half(x, x)
    ot_xx = _ot_from_C(C_xx)
    ii = jax.lax.broadcasted_iota(jnp.int32, (N, N), 0)
    jj = jax.lax.broadcasted_iota(jnp.int32, (N, N), 1)
    off = ii != jj
    usum = jnp.sum(jnp.where(off, jnp.exp(-4.0 * C_xx), 0.0))
    uni = jnp.log(usum / (N * (N - 1)))
    C_yy = _sqd_half(y, y)
    ot_yy = _ot_from_C(C_yy)
    out_ref[0, 0] = ot_xy - 0.5 * (ot_xx + ot_yy) + uni


@jax.jit
def kernel(P_r, P_gt, pose_gt, map_pts):
    pts = map_pts[:, :3]
    mapT = pts.T
    mx = pts[:, 0].reshape(512, 128)
    my = pts[:, 1].reshape(512, 128)
    mz = pts[:, 2].reshape(512, 128)

    tri = pl.pallas_call(
        _tri_kernel,
        out_shape=jax.ShapeDtypeStruct((1, 1), jnp.float32),
        in_specs=[
            pl.BlockSpec(memory_space=pltpu.VMEM),
            pl.BlockSpec(memory_space=pltpu.VMEM),
            pl.BlockSpec(memory_space=pltpu.VMEM),
            pl.BlockSpec(memory_space=pltpu.VMEM),
            pl.BlockSpec(memory_space=pltpu.VMEM),
            pl.BlockSpec(memory_space=pltpu.SMEM),
        ],
        out_specs=pl.BlockSpec(memory_space=pltpu.SMEM),
        scratch_shapes=[
            pltpu.VMEM((NCHUNK, CHUNK), jnp.float32),
            pltpu.VMEM((NCHUNK, CHUNK), jnp.float32),
        ],
    )
    up = pl.pallas_call(
        _upsample_kernel,
        out_shape=jax.ShapeDtypeStruct((1, 1), jnp.float32),
        out_specs=pl.BlockSpec(memory_space=pltpu.SMEM),
    )
    l_tri = tri(P_r, mapT, mx, my, mz, pose_gt)[0, 0]
    l_up = up(P_r, P_gt)[0, 0]
    return l_up * RATIO + l_tri * (1.0 - RATIO)
